# baseline (speedup 1.0000x reference)
import functools

import jax
import jax.numpy as jnp
import numpy as np
from jax import lax
from jax.experimental import pallas as pl
from jax.experimental.pallas import tpu as pltpu
from jax.experimental.pallas import tpu_sc as plsc

_EMBED_DIM = 1024
_MAX_LENGTH = 4096
_SEQ_LEN = 4096

_NUM_CORES = 2
_NUM_SUBCORES = 16
_NUM_WORKERS = _NUM_CORES * _NUM_SUBCORES
_LANES = 16
_COLS = _EMBED_DIM // _LANES

_SC_ROWS = 1536
_TC_ROWS = _SEQ_LEN - _SC_ROWS


def _pe_shifted_np() -> np.ndarray:
    d_model, max_len = _EMBED_DIM, _MAX_LENGTH
    position = np.arange(max_len, dtype=np.float64)[:, None]
    div_term = np.exp(
        np.arange(0, d_model, 2, dtype=np.float64) * (-np.log(10000.0) / d_model)
    )
    pe = np.zeros((max_len, d_model), dtype=np.float64)
    pe[:, 0::2] = np.sin(position * div_term)
    pe[:, 1::2] = np.cos(position * div_term)
    out = np.zeros((_SEQ_LEN, d_model), dtype=np.float64)
    out[1:] = pe[: _SEQ_LEN - 1]
    return out.astype(np.float32)


_PE_SHIFTED = _pe_shifted_np()


def _sc_body(bin_hbm, table_hbm, pe_hbm, out_hbm,
             bin_v, tab_v, buf0, buf1, sem_pe0, sem_pe1, sem_o0, sem_o1,
             *, rows_per_worker, chunk):
    steps = rows_per_worker // chunk
    wid = lax.axis_index("s") * _NUM_CORES + lax.axis_index("c")
    base = wid * rows_per_worker

    pltpu.sync_copy(bin_hbm.at[pl.ds(base, rows_per_worker)], bin_v)
    pltpu.sync_copy(table_hbm, tab_v)

    bufs = (buf0, buf1)
    pe_sems = (sem_pe0, sem_pe1)
    out_sems = (sem_o0, sem_o1)

    def step_compute(buf, t):
        for g in range(chunk // 16):
            bv = bin_v[pl.ds(t * chunk + g * 16, 16)].astype(jnp.float32)
            bfs = [jnp.full((16,), bv[r]) for r in range(16)]

            def col_body(c, carry, _bfs=bfs, _g=g):
                sl = pl.ds(c * _LANES, _LANES)
                e0 = tab_v[0, sl]
                d = tab_v[1, sl] - e0
                for r in range(16):
                    row = _g * 16 + r
                    buf[row, sl] = (buf[row, sl] + e0) + _bfs[r] * d
                return carry

            lax.fori_loop(0, _COLS, col_body, 0)

    cp_pe = [None] * steps
    cp_out = [None] * steps
    cp_pe[0] = pltpu.async_copy(
        pe_hbm.at[pl.ds(base, chunk)], bufs[0], pe_sems[0])
    for t in range(steps):
        p = t & 1
        if t + 1 < steps:
            if t >= 1:
                cp_out[t - 1].wait()
            cp_pe[t + 1] = pltpu.async_copy(
                pe_hbm.at[pl.ds(base + (t + 1) * chunk, chunk)],
                bufs[1 - p], pe_sems[1 - p])
        cp_pe[t].wait()
        step_compute(bufs[p], t)
        cp_out[t] = pltpu.async_copy(
            bufs[p], out_hbm.at[pl.ds(base + t * chunk, chunk)], out_sems[p])
    cp_out[steps - 2].wait()
    cp_out[steps - 1].wait()


def _encode_sc(binary, embed_table, pe, n_rows, chunk):
    rows_per_worker = n_rows // _NUM_WORKERS
    mesh = plsc.VectorSubcoreMesh(core_axis_name="c", subcore_axis_name="s")
    body = functools.partial(
        _sc_body, rows_per_worker=rows_per_worker, chunk=chunk)
    f = pl.kernel(
        body,
        mesh=mesh,
        out_type=jax.ShapeDtypeStruct((n_rows, _EMBED_DIM), jnp.float32),
        scratch_types=[
            pltpu.VMEM((rows_per_worker,), jnp.int32),
            pltpu.VMEM((2, _EMBED_DIM), jnp.float32),
            pltpu.VMEM((chunk, _EMBED_DIM), jnp.float32),
            pltpu.VMEM((chunk, _EMBED_DIM), jnp.float32),
            pltpu.SemaphoreType.DMA,
            pltpu.SemaphoreType.DMA,
            pltpu.SemaphoreType.DMA,
            pltpu.SemaphoreType.DMA,
        ],
    )
    return f(binary, embed_table, pe)


_TC_BLOCK = 512
_TC_GRID = _TC_ROWS // _TC_BLOCK


def _tc_body(bin_ref, tab_ref, pe_ref, out_ref):
    b = bin_ref[...].astype(jnp.float32)
    e0 = tab_ref[0, :][None, :]
    d = tab_ref[1, :][None, :] - e0
    out_ref[...] = pe_ref[...] + e0 + b * d


def _encode_tc(binary_tc, embed_table, pe_tc):
    return pl.pallas_call(
        _tc_body,
        grid=(_TC_GRID,),
        in_specs=[
            pl.BlockSpec((_TC_BLOCK, 1), lambda i: (i, 0)),
            pl.BlockSpec((2, _EMBED_DIM), lambda i: (0, 0)),
            pl.BlockSpec((_TC_BLOCK, _EMBED_DIM), lambda i: (i, 0)),
        ],
        out_specs=pl.BlockSpec((_TC_BLOCK, _EMBED_DIM), lambda i: (i, 0)),
        out_shape=jax.ShapeDtypeStruct((_TC_ROWS, _EMBED_DIM), jnp.float32),
    )(binary_tc.reshape(_TC_ROWS, 1), embed_table, pe_tc)


@jax.jit
def _encode(binary, embed_table, pe):
    out_sc = _encode_sc(binary[:_SC_ROWS], embed_table, pe[:_SC_ROWS],
                        _SC_ROWS, 16)
    out_tc = _encode_tc(binary[_SC_ROWS:], embed_table, pe[_SC_ROWS:])
    return jnp.concatenate([out_sc, out_tc], axis=0)


def kernel(binary, embed_table):
    pe = jnp.asarray(_PE_SHIFTED)
    return _encode(binary, embed_table, pe)


# speedup vs baseline: 1.0037x; 1.0037x over previous
import functools

import jax
import jax.numpy as jnp
import numpy as np
from jax import lax
from jax.experimental import pallas as pl
from jax.experimental.pallas import tpu as pltpu
from jax.experimental.pallas import tpu_sc as plsc

_EMBED_DIM = 1024
_MAX_LENGTH = 4096
_SEQ_LEN = 4096

_NUM_CORES = 2
_NUM_SUBCORES = 16
_NUM_WORKERS = _NUM_CORES * _NUM_SUBCORES
_LANES = 16
_COLS = _EMBED_DIM // _LANES

_SC_ROWS = 1536
_TC_ROWS = _SEQ_LEN - _SC_ROWS


def _pe_shifted_np() -> np.ndarray:
    d_model, max_len = _EMBED_DIM, _MAX_LENGTH
    position = np.arange(max_len, dtype=np.float64)[:, None]
    div_term = np.exp(
        np.arange(0, d_model, 2, dtype=np.float64) * (-np.log(10000.0) / d_model)
    )
    pe = np.zeros((max_len, d_model), dtype=np.float64)
    pe[:, 0::2] = np.sin(position * div_term)
    pe[:, 1::2] = np.cos(position * div_term)
    out = np.zeros((_SEQ_LEN, d_model), dtype=np.float64)
    out[1:] = pe[: _SEQ_LEN - 1]
    return out.astype(np.float32)


_PE_SHIFTED = _pe_shifted_np()


def _sc_body(bin_hbm, table_hbm, pe_hbm, out_hbm,
             bin_v, tab_v, buf0, buf1, sem_pe0, sem_pe1, sem_o0, sem_o1,
             *, rows_per_worker, chunk):
    steps = rows_per_worker // chunk
    wid = lax.axis_index("s") * _NUM_CORES + lax.axis_index("c")
    base = wid * rows_per_worker

    pltpu.sync_copy(bin_hbm.at[pl.ds(base, rows_per_worker)], bin_v)
    pltpu.sync_copy(table_hbm, tab_v)

    bufs = (buf0, buf1)
    pe_sems = (sem_pe0, sem_pe1)
    out_sems = (sem_o0, sem_o1)

    def step_compute(buf, t):
        for g in range(chunk // 16):
            bv = bin_v[pl.ds(t * chunk + g * 16, 16)].astype(jnp.float32)
            bfs = [jnp.full((16,), bv[r]) for r in range(16)]

            def col_body(c, carry, _bfs=bfs, _g=g):
                sl = pl.ds(c * _LANES, _LANES)
                e0 = tab_v[0, sl]
                d = tab_v[1, sl] - e0
                for r in range(16):
                    row = _g * 16 + r
                    buf[row, sl] = (buf[row, sl] + e0) + _bfs[r] * d
                return carry

            lax.fori_loop(0, _COLS, col_body, 0)

    cp_pe = [None] * steps
    cp_out = [None] * steps
    cp_pe[0] = pltpu.async_copy(
        pe_hbm.at[pl.ds(base, chunk)], bufs[0], pe_sems[0])
    for t in range(steps):
        p = t & 1
        if t + 1 < steps:
            if t >= 1:
                cp_out[t - 1].wait()
            cp_pe[t + 1] = pltpu.async_copy(
                pe_hbm.at[pl.ds(base + (t + 1) * chunk, chunk)],
                bufs[1 - p], pe_sems[1 - p])
        cp_pe[t].wait()
        step_compute(bufs[p], t)
        cp_out[t] = pltpu.async_copy(
            bufs[p], out_hbm.at[pl.ds(base + t * chunk, chunk)], out_sems[p])
    cp_out[steps - 2].wait()
    cp_out[steps - 1].wait()


def _encode_sc(binary, embed_table, pe, n_rows, chunk):
    rows_per_worker = n_rows // _NUM_WORKERS
    mesh = plsc.VectorSubcoreMesh(core_axis_name="c", subcore_axis_name="s")
    body = functools.partial(
        _sc_body, rows_per_worker=rows_per_worker, chunk=chunk)
    f = pl.kernel(
        body,
        mesh=mesh,
        out_type=jax.ShapeDtypeStruct((n_rows, _EMBED_DIM), jnp.float32),
        scratch_types=[
            pltpu.VMEM((rows_per_worker,), jnp.int32),
            pltpu.VMEM((2, _EMBED_DIM), jnp.float32),
            pltpu.VMEM((chunk, _EMBED_DIM), jnp.float32),
            pltpu.VMEM((chunk, _EMBED_DIM), jnp.float32),
            pltpu.SemaphoreType.DMA,
            pltpu.SemaphoreType.DMA,
            pltpu.SemaphoreType.DMA,
            pltpu.SemaphoreType.DMA,
        ],
    )
    return f(binary, embed_table, pe)


_TC_BLOCK = 512
_TC_GRID = _TC_ROWS // _TC_BLOCK


_TC_BLOCK0 = _SC_ROWS // _TC_BLOCK


def _tc_body(bin_ref, tab_ref, pe_ref, out_ref):
    b = bin_ref[...].astype(jnp.float32)
    e0 = tab_ref[0, :][None, :]
    d = tab_ref[1, :][None, :] - e0
    out_ref[...] = pe_ref[...] + e0 + b * d


def _encode_tc(binary, embed_table, pe):
    return pl.pallas_call(
        _tc_body,
        grid=(_TC_GRID,),
        in_specs=[
            pl.BlockSpec((_TC_BLOCK, 1), lambda i: (i + _TC_BLOCK0, 0)),
            pl.BlockSpec((2, _EMBED_DIM), lambda i: (0, 0)),
            pl.BlockSpec((_TC_BLOCK, _EMBED_DIM), lambda i: (i + _TC_BLOCK0, 0)),
        ],
        out_specs=pl.BlockSpec((_TC_BLOCK, _EMBED_DIM),
                               lambda i: (i + _TC_BLOCK0, 0)),
        out_shape=jax.ShapeDtypeStruct((_SEQ_LEN, _EMBED_DIM), jnp.float32),
    )(binary.reshape(_SEQ_LEN, 1), embed_table, pe)


@jax.jit
def _encode(binary, embed_table, pe):
    out_sc = _encode_sc(binary, embed_table, pe, _SC_ROWS, 16)
    out_tc = _encode_tc(binary, embed_table, pe)
    return lax.dynamic_update_slice(out_tc, out_sc, (0, 0))


def kernel(binary, embed_table):
    pe = jnp.asarray(_PE_SHIFTED)
    return _encode(binary, embed_table, pe)


# speedup vs baseline: 1.1143x; 1.1102x over previous
import functools

import jax
import jax.numpy as jnp
import numpy as np
from jax import lax
from jax.experimental import pallas as pl
from jax.experimental.pallas import tpu as pltpu
from jax.experimental.pallas import tpu_sc as plsc

_EMBED_DIM = 1024
_MAX_LENGTH = 4096
_SEQ_LEN = 4096

_NUM_CORES = 2
_NUM_SUBCORES = 16
_NUM_WORKERS = _NUM_CORES * _NUM_SUBCORES
_LANES = 16
_COLS = _EMBED_DIM // _LANES

_SC_ROWS = 1024
_TC_ROWS = _SEQ_LEN - _SC_ROWS


def _pe_shifted_np() -> np.ndarray:
    d_model, max_len = _EMBED_DIM, _MAX_LENGTH
    position = np.arange(max_len, dtype=np.float64)[:, None]
    div_term = np.exp(
        np.arange(0, d_model, 2, dtype=np.float64) * (-np.log(10000.0) / d_model)
    )
    pe = np.zeros((max_len, d_model), dtype=np.float64)
    pe[:, 0::2] = np.sin(position * div_term)
    pe[:, 1::2] = np.cos(position * div_term)
    out = np.zeros((_SEQ_LEN, d_model), dtype=np.float64)
    out[1:] = pe[: _SEQ_LEN - 1]
    return out.astype(np.float32)


_PE_SHIFTED = _pe_shifted_np()


def _sc_body(bin_hbm, table_hbm, pe_hbm, out_hbm,
             bin_v, tab_v, buf0, buf1, sem_pe0, sem_pe1, sem_o0, sem_o1,
             *, rows_per_worker, chunk):
    steps = rows_per_worker // chunk
    wid = lax.axis_index("s") * _NUM_CORES + lax.axis_index("c")
    base = wid * rows_per_worker

    pltpu.sync_copy(bin_hbm.at[pl.ds(base, rows_per_worker)], bin_v)
    pltpu.sync_copy(table_hbm, tab_v)

    bufs = (buf0, buf1)
    pe_sems = (sem_pe0, sem_pe1)
    out_sems = (sem_o0, sem_o1)

    def step_compute(buf, t):
        for g in range(chunk // 16):
            bv = bin_v[pl.ds(t * chunk + g * 16, 16)].astype(jnp.float32)
            bfs = [jnp.full((16,), bv[r]) for r in range(16)]

            def col_body(c, carry, _bfs=bfs, _g=g):
                sl = pl.ds(c * _LANES, _LANES)
                e0 = tab_v[0, sl]
                d = tab_v[1, sl] - e0
                for r in range(16):
                    row = _g * 16 + r
                    buf[row, sl] = (buf[row, sl] + e0) + _bfs[r] * d
                return carry

            lax.fori_loop(0, _COLS, col_body, 0)

    cp_pe = [None] * steps
    cp_out = [None] * steps
    cp_pe[0] = pltpu.async_copy(
        pe_hbm.at[pl.ds(base, chunk)], bufs[0], pe_sems[0])
    for t in range(steps):
        p = t & 1
        if t + 1 < steps:
            if t >= 1:
                cp_out[t - 1].wait()
            cp_pe[t + 1] = pltpu.async_copy(
                pe_hbm.at[pl.ds(base + (t + 1) * chunk, chunk)],
                bufs[1 - p], pe_sems[1 - p])
        cp_pe[t].wait()
        step_compute(bufs[p], t)
        cp_out[t] = pltpu.async_copy(
            bufs[p], out_hbm.at[pl.ds(base + t * chunk, chunk)], out_sems[p])
    cp_out[steps - 2].wait()
    cp_out[steps - 1].wait()


def _encode_sc(binary, embed_table, pe, n_rows, chunk):
    rows_per_worker = n_rows // _NUM_WORKERS
    mesh = plsc.VectorSubcoreMesh(core_axis_name="c", subcore_axis_name="s")
    body = functools.partial(
        _sc_body, rows_per_worker=rows_per_worker, chunk=chunk)
    f = pl.kernel(
        body,
        mesh=mesh,
        out_type=jax.ShapeDtypeStruct((n_rows, _EMBED_DIM), jnp.float32),
        scratch_types=[
            pltpu.VMEM((rows_per_worker,), jnp.int32),
            pltpu.VMEM((2, _EMBED_DIM), jnp.float32),
            pltpu.VMEM((chunk, _EMBED_DIM), jnp.float32),
            pltpu.VMEM((chunk, _EMBED_DIM), jnp.float32),
            pltpu.SemaphoreType.DMA,
            pltpu.SemaphoreType.DMA,
            pltpu.SemaphoreType.DMA,
            pltpu.SemaphoreType.DMA,
        ],
    )
    return f(binary, embed_table, pe)


_TC_BLOCK = 512
_TC_GRID = _TC_ROWS // _TC_BLOCK


_TC_BLOCK0 = _SC_ROWS // _TC_BLOCK


def _tc_body(bin_ref, tab_ref, pe_ref, out_ref):
    b = bin_ref[...].astype(jnp.float32)
    onehot = jnp.concatenate([1.0 - b, b], axis=1)
    out_ref[...] = pe_ref[...] + jnp.dot(
        onehot, tab_ref[...], preferred_element_type=jnp.float32,
        precision=lax.Precision.HIGHEST)


def _encode_tc(binary, embed_table, pe):
    return pl.pallas_call(
        _tc_body,
        grid=(_TC_GRID,),
        in_specs=[
            pl.BlockSpec((_TC_BLOCK, 1), lambda i: (i + _TC_BLOCK0, 0)),
            pl.BlockSpec((2, _EMBED_DIM), lambda i: (0, 0)),
            pl.BlockSpec((_TC_BLOCK, _EMBED_DIM), lambda i: (i + _TC_BLOCK0, 0)),
        ],
        out_specs=pl.BlockSpec((_TC_BLOCK, _EMBED_DIM),
                               lambda i: (i + _TC_BLOCK0, 0)),
        out_shape=jax.ShapeDtypeStruct((_SEQ_LEN, _EMBED_DIM), jnp.float32),
    )(binary.reshape(_SEQ_LEN, 1), embed_table, pe)


@jax.jit
def _encode(binary, embed_table, pe):
    out_sc = _encode_sc(binary[:_SC_ROWS], embed_table, pe[:_SC_ROWS],
                        _SC_ROWS, 16)
    out_tc = _encode_tc(binary, embed_table, pe)
    return lax.dynamic_update_slice(out_tc, out_sc, (0, 0))


def kernel(binary, embed_table):
    pe = jnp.asarray(_PE_SHIFTED)
    return _encode(binary, embed_table, pe)


# speedup vs baseline: 1.1640x; 1.0446x over previous
import functools

import jax
import jax.numpy as jnp
import numpy as np
from jax import lax
from jax.experimental import pallas as pl
from jax.experimental.pallas import tpu as pltpu
from jax.experimental.pallas import tpu_sc as plsc

_EMBED_DIM = 1024
_MAX_LENGTH = 4096
_SEQ_LEN = 4096

_NUM_CORES = 2
_NUM_SUBCORES = 16
_NUM_WORKERS = _NUM_CORES * _NUM_SUBCORES
_LANES = 16
_COLS = _EMBED_DIM // _LANES

_SC_ROWS = 1024
_TC_ROWS = _SEQ_LEN - _SC_ROWS


def _pe_shifted_np() -> np.ndarray:
    d_model, max_len = _EMBED_DIM, _MAX_LENGTH
    position = np.arange(max_len, dtype=np.float64)[:, None]
    div_term = np.exp(
        np.arange(0, d_model, 2, dtype=np.float64) * (-np.log(10000.0) / d_model)
    )
    pe = np.zeros((max_len, d_model), dtype=np.float64)
    pe[:, 0::2] = np.sin(position * div_term)
    pe[:, 1::2] = np.cos(position * div_term)
    out = np.zeros((_SEQ_LEN, d_model), dtype=np.float64)
    out[1:] = pe[: _SEQ_LEN - 1]
    return out.astype(np.float32)


_PE_SHIFTED = _pe_shifted_np()


def _sc_body(bin_hbm, table_hbm, pe_hbm, out_hbm,
             bin_v, tab_v, buf0, buf1, sem_pe0, sem_pe1, sem_o0, sem_o1,
             *, rows_per_worker, chunk):
    steps = rows_per_worker // chunk
    wid = lax.axis_index("s") * _NUM_CORES + lax.axis_index("c")
    base = wid * rows_per_worker

    pltpu.sync_copy(bin_hbm.at[pl.ds(base, rows_per_worker)], bin_v)
    pltpu.sync_copy(table_hbm, tab_v)

    bufs = (buf0, buf1)
    pe_sems = (sem_pe0, sem_pe1)
    out_sems = (sem_o0, sem_o1)

    def step_compute(buf, t):
        for g in range(chunk // 16):
            bv = bin_v[pl.ds(t * chunk + g * 16, 16)].astype(jnp.float32)
            bfs = [jnp.full((16,), bv[r]) for r in range(16)]

            def col_body(c, carry, _bfs=bfs, _g=g):
                sl = pl.ds(c * _LANES, _LANES)
                e0 = tab_v[0, sl]
                d = tab_v[1, sl] - e0
                for r in range(16):
                    row = _g * 16 + r
                    buf[row, sl] = (buf[row, sl] + e0) + _bfs[r] * d
                return carry

            lax.fori_loop(0, _COLS, col_body, 0)

    cp_pe = [None] * steps
    cp_out = [None] * steps
    cp_pe[0] = pltpu.async_copy(
        pe_hbm.at[pl.ds(base, chunk)], bufs[0], pe_sems[0])
    for t in range(steps):
        p = t & 1
        if t + 1 < steps:
            if t >= 1:
                cp_out[t - 1].wait()
            cp_pe[t + 1] = pltpu.async_copy(
                pe_hbm.at[pl.ds(base + (t + 1) * chunk, chunk)],
                bufs[1 - p], pe_sems[1 - p])
        cp_pe[t].wait()
        step_compute(bufs[p], t)
        cp_out[t] = pltpu.async_copy(
            bufs[p], out_hbm.at[pl.ds(base + t * chunk, chunk)], out_sems[p])
    cp_out[steps - 2].wait()
    cp_out[steps - 1].wait()


def _encode_sc(binary, embed_table, pe, n_rows, chunk):
    rows_per_worker = n_rows // _NUM_WORKERS
    mesh = plsc.VectorSubcoreMesh(core_axis_name="c", subcore_axis_name="s")
    body = functools.partial(
        _sc_body, rows_per_worker=rows_per_worker, chunk=chunk)
    f = pl.kernel(
        body,
        mesh=mesh,
        out_type=jax.ShapeDtypeStruct((n_rows, _EMBED_DIM), jnp.float32),
        scratch_types=[
            pltpu.VMEM((rows_per_worker,), jnp.int32),
            pltpu.VMEM((2, _EMBED_DIM), jnp.float32),
            pltpu.VMEM((chunk, _EMBED_DIM), jnp.float32),
            pltpu.VMEM((chunk, _EMBED_DIM), jnp.float32),
            pltpu.SemaphoreType.DMA,
            pltpu.SemaphoreType.DMA,
            pltpu.SemaphoreType.DMA,
            pltpu.SemaphoreType.DMA,
        ],
    )
    return f(binary, embed_table, pe)


_TC_BLOCK = 256
_TC_GRID = _TC_ROWS // _TC_BLOCK


_TC_BLOCK0 = _SC_ROWS // _TC_BLOCK


def _tc_body(bin_ref, tab_ref, pe_ref, out_ref):
    b = bin_ref[...].astype(jnp.float32)
    e0 = tab_ref[0, :][None, :]
    d = tab_ref[1, :][None, :] - e0
    out_ref[...] = pe_ref[...] + e0 + b * d


def _encode_tc(binary, embed_table, pe):
    return pl.pallas_call(
        _tc_body,
        grid=(_TC_GRID,),
        in_specs=[
            pl.BlockSpec((_TC_BLOCK, 1), lambda i: (i + _TC_BLOCK0, 0)),
            pl.BlockSpec((2, _EMBED_DIM), lambda i: (0, 0)),
            pl.BlockSpec((_TC_BLOCK, _EMBED_DIM), lambda i: (i + _TC_BLOCK0, 0)),
        ],
        out_specs=pl.BlockSpec((_TC_BLOCK, _EMBED_DIM),
                               lambda i: (i + _TC_BLOCK0, 0)),
        out_shape=jax.ShapeDtypeStruct((_SEQ_LEN, _EMBED_DIM), jnp.float32),
    )(binary.reshape(_SEQ_LEN, 1), embed_table, pe)


@jax.jit
def _encode(binary, embed_table, pe_sc, pe):
    out_sc = _encode_sc(binary[:_SC_ROWS], embed_table, pe_sc, _SC_ROWS, 16)
    out_tc = _encode_tc(binary, embed_table, pe)
    return lax.dynamic_update_slice(out_tc, out_sc, (0, 0))


def kernel(binary, embed_table):
    pe_sc = jnp.asarray(_PE_SHIFTED[:_SC_ROWS])
    pe = jnp.asarray(_PE_SHIFTED)
    return _encode(binary, embed_table, pe_sc, pe)
